# baseline (speedup 1.0000x reference)
import functools
import math

import jax
import jax.numpy as jnp
import numpy as np
from jax.experimental import pallas as pl

_LEVELS = [15, 15, 15]
_C = len(_LEVELS)
_CP = 128
_EPS = 1e-3

_HALF_L = (_LEVELS[0] - 1.0) * (1.0 + _EPS) / 2.0
_OFFSET = 0.5 if _LEVELS[0] % 2 == 0 else 0.0
_SHIFT = math.atanh(_OFFSET / _HALF_L)
_HALF_WIDTH = float(np.floor(_LEVELS[0] / 2.0))


def _fsq_kernel(z_ref, wc_ref, bc_ref, we_ref, be_ref, traw_ref,
                zq_ref, err_ref, *, n_valid):
    i = pl.program_id(0)

    t = jax.nn.softplus(traw_ref[...])
    inv_t = 1.0 / t
    scale = t * (1.0 / _HALF_WIDTH)

    zc = jax.lax.dot_general(
        z_ref[...], wc_ref[...], (((1,), (0,)), ((), ())),
        preferred_element_type=jnp.float32,
        precision=jax.lax.Precision.DEFAULT,
    ) + bc_ref[...]

    zb = jnp.tanh(zc * inv_t + _SHIFT) * _HALF_L - _OFFSET
    codes = jnp.round(zb) * scale

    d = zc - codes
    part = (jnp.sum(d * d) * (1.0 / n_valid)).reshape(1, 1)

    @pl.when(i == 0)
    def _():
        err_ref[...] = jnp.zeros((1, 1), jnp.float32)

    err_ref[...] += part

    zq_ref[...] = jax.lax.dot_general(
        codes, we_ref[...], (((1,), (0,)), ((), ())),
        preferred_element_type=jnp.float32,
        precision=jax.lax.Precision.DEFAULT,
    ) + be_ref[...]


def kernel(z, W_c, b_c, W_e, b_e, T_raw):
    B, S, D = z.shape
    N = B * S
    z2 = z.reshape(N, D)

    wc_t = jnp.zeros((D, _CP), jnp.float32).at[:, :_C].set(W_c.T)
    bc = jnp.zeros((1, _CP), jnp.float32).at[0, :_C].set(b_c)
    we_t = jnp.zeros((_CP, D), jnp.float32).at[:_C, :].set(W_e.T)
    be = b_e.reshape(1, D).astype(jnp.float32)
    traw = jnp.zeros((1, _CP), jnp.float32).at[0, :_C].set(T_raw)

    R = 2048
    grid = (N // R,)

    zq, err = pl.pallas_call(
        functools.partial(_fsq_kernel, n_valid=float(N * _C)),
        grid=grid,
        in_specs=[
            pl.BlockSpec((R, D), lambda i: (i, 0)),
            pl.BlockSpec((D, _CP), lambda i: (0, 0)),
            pl.BlockSpec((1, _CP), lambda i: (0, 0)),
            pl.BlockSpec((_CP, D), lambda i: (0, 0)),
            pl.BlockSpec((1, D), lambda i: (0, 0)),
            pl.BlockSpec((1, _CP), lambda i: (0, 0)),
        ],
        out_specs=[
            pl.BlockSpec((R, D), lambda i: (i, 0)),
            pl.BlockSpec((1, 1), lambda i: (0, 0)),
        ],
        out_shape=[
            jax.ShapeDtypeStruct((N, D), jnp.float32),
            jax.ShapeDtypeStruct((1, 1), jnp.float32),
        ],
    )(z2, wc_t, bc, we_t, be, traw)

    return zq.reshape(B, S, D), err[0, 0]


# speedup vs baseline: 1.0540x; 1.0540x over previous
import functools
import math

import jax
import jax.numpy as jnp
import numpy as np
from jax.experimental import pallas as pl

_LEVELS = [15, 15, 15]
_C = len(_LEVELS)
_CP = 128
_EPS = 1e-3

_HALF_L = (_LEVELS[0] - 1.0) * (1.0 + _EPS) / 2.0
_OFFSET = 0.5 if _LEVELS[0] % 2 == 0 else 0.0
_SHIFT = math.atanh(_OFFSET / _HALF_L)
_HALF_WIDTH = float(np.floor(_LEVELS[0] / 2.0))


def _fsq_kernel(z_ref, wc_ref, bc_ref, we_ref, be_ref, traw_ref,
                zq_ref, err_ref, *, n_valid):
    i = pl.program_id(0)

    t = jax.nn.softplus(traw_ref[...])
    inv_t = 1.0 / t
    scale = t * (1.0 / _HALF_WIDTH)

    zc = jax.lax.dot_general(
        z_ref[...], wc_ref[...], (((1,), (0,)), ((), ())),
        preferred_element_type=jnp.float32,
        precision=jax.lax.Precision.DEFAULT,
    ) + bc_ref[...]

    zb = jnp.tanh(zc * inv_t + _SHIFT) * _HALF_L - _OFFSET
    codes = jnp.round(zb) * scale

    d = zc - codes
    part = (jnp.sum(d * d) * (1.0 / n_valid)).reshape(1, 1)

    @pl.when(i == 0)
    def _():
        err_ref[...] = jnp.zeros((1, 1), jnp.float32)

    err_ref[...] += part

    acc = codes[:, 0:1] * we_ref[0:1, :]
    acc = acc + codes[:, 1:2] * we_ref[1:2, :]
    acc = acc + codes[:, 2:3] * we_ref[2:3, :]
    zq_ref[...] = acc + be_ref[...]


def kernel(z, W_c, b_c, W_e, b_e, T_raw):
    B, S, D = z.shape
    N = B * S
    z2 = z.reshape(N, D)

    wc_t = jnp.zeros((D, _CP), jnp.float32).at[:, :_C].set(W_c.T)
    bc = jnp.zeros((1, _CP), jnp.float32).at[0, :_C].set(b_c)
    we_t = jnp.zeros((8, D), jnp.float32).at[:_C, :].set(W_e.T)
    be = b_e.reshape(1, D).astype(jnp.float32)
    traw = jnp.zeros((1, _CP), jnp.float32).at[0, :_C].set(T_raw)

    R = 2048
    grid = (N // R,)

    zq, err = pl.pallas_call(
        functools.partial(_fsq_kernel, n_valid=float(N * _C)),
        grid=grid,
        in_specs=[
            pl.BlockSpec((R, D), lambda i: (i, 0)),
            pl.BlockSpec((D, _CP), lambda i: (0, 0)),
            pl.BlockSpec((1, _CP), lambda i: (0, 0)),
            pl.BlockSpec((8, D), lambda i: (0, 0)),
            pl.BlockSpec((1, D), lambda i: (0, 0)),
            pl.BlockSpec((1, _CP), lambda i: (0, 0)),
        ],
        out_specs=[
            pl.BlockSpec((R, D), lambda i: (i, 0)),
            pl.BlockSpec((1, 1), lambda i: (0, 0)),
        ],
        out_shape=[
            jax.ShapeDtypeStruct((N, D), jnp.float32),
            jax.ShapeDtypeStruct((1, 1), jnp.float32),
        ],
    )(z2, wc_t, bc, we_t, be, traw)

    return zq.reshape(B, S, D), err[0, 0]


# speedup vs baseline: 1.1066x; 1.0499x over previous
import functools
import math

import jax
import jax.numpy as jnp
import numpy as np
from jax.experimental import pallas as pl

_LEVELS = [15, 15, 15]
_C = len(_LEVELS)
_CP = 128
_EPS = 1e-3

_HALF_L = (_LEVELS[0] - 1.0) * (1.0 + _EPS) / 2.0
_OFFSET = 0.5 if _LEVELS[0] % 2 == 0 else 0.0
_SHIFT = math.atanh(_OFFSET / _HALF_L)
_HALF_WIDTH = float(np.floor(_LEVELS[0] / 2.0))


def _fsq_kernel(z_ref, wc_ref, bc_ref, we_ref, be_ref, traw_ref,
                zq_ref, err_ref, *, n_valid):
    i = pl.program_id(0)

    t = jax.nn.softplus(traw_ref[...])
    inv_t = 1.0 / t
    scale = t * (1.0 / _HALF_WIDTH)

    zc = jax.lax.dot_general(
        z_ref[...], wc_ref[...], (((1,), (0,)), ((), ())),
        preferred_element_type=jnp.float32,
        precision=jax.lax.Precision.DEFAULT,
    ) + bc_ref[...]

    zb = jnp.tanh(zc * inv_t + _SHIFT) * _HALF_L - _OFFSET
    codes = jnp.round(zb) * scale

    d = zc - codes
    part = (jnp.sum(d * d) * (1.0 / n_valid)).reshape(1, 1)

    @pl.when(i == 0)
    def _():
        err_ref[...] = jnp.zeros((1, 1), jnp.float32)

    err_ref[...] += part

    acc = codes[:, 0:1] * we_ref[0:1, :]
    acc = acc + codes[:, 1:2] * we_ref[1:2, :]
    acc = acc + codes[:, 2:3] * we_ref[2:3, :]
    zq_ref[...] = acc + be_ref[...]


def kernel(z, W_c, b_c, W_e, b_e, T_raw):
    B, S, D = z.shape
    N = B * S
    z2 = z.reshape(N, D)

    wc_t = jnp.zeros((D, _CP), jnp.bfloat16).at[:, :_C].set(W_c.T.astype(jnp.bfloat16))
    bc = jnp.zeros((1, _CP), jnp.float32).at[0, :_C].set(b_c)
    we_t = jnp.zeros((8, D), jnp.float32).at[:_C, :].set(W_e.T)
    be = b_e.reshape(1, D).astype(jnp.float32)
    traw = jnp.zeros((1, _CP), jnp.float32).at[0, :_C].set(T_raw)

    R = 2048
    grid = (N // R,)

    zq, err = pl.pallas_call(
        functools.partial(_fsq_kernel, n_valid=float(N * _C)),
        grid=grid,
        in_specs=[
            pl.BlockSpec((R, D), lambda i: (i, 0)),
            pl.BlockSpec((D, _CP), lambda i: (0, 0)),
            pl.BlockSpec((1, _CP), lambda i: (0, 0)),
            pl.BlockSpec((8, D), lambda i: (0, 0)),
            pl.BlockSpec((1, D), lambda i: (0, 0)),
            pl.BlockSpec((1, _CP), lambda i: (0, 0)),
        ],
        out_specs=[
            pl.BlockSpec((R, D), lambda i: (i, 0)),
            pl.BlockSpec((1, 1), lambda i: (0, 0)),
        ],
        out_shape=[
            jax.ShapeDtypeStruct((N, D), jnp.float32),
            jax.ShapeDtypeStruct((1, 1), jnp.float32),
        ],
    )(z2, wc_t, bc, we_t, be, traw)

    return zq.reshape(B, S, D), err[0, 0]
